# baseline (speedup 1.0000x reference)
import functools

import jax
import jax.numpy as jnp
from jax import lax
from jax.experimental import pallas as pl
from jax.experimental.pallas import tpu as pltpu
from jax.experimental.pallas import tpu_sc as plsc

N = 10000
E = 320000
HEADS = 8
HDIM = 16
NG = 100
ROWS = 56
NC, NS = 2, 16
EPW = E // (NC * NS)
RB = 80
CH = 2000
RJ = CH // RB
NCHUNK = EPW // CH


def _me_body(msg_ref, wfc_ref, bfc_ref, out_ref):
    out_ref[...] = (
        jnp.dot(msg_ref[...], wfc_ref[...], preferred_element_type=jnp.float32)
        + bfc_ref[...]
    )


def _t_body(x_ref, w_ref, g_ref, d_ref, out_ref):
    blk = x_ref.shape[0]
    half = ROWS * 16
    wh = jnp.dot(x_ref[...], w_ref[...], preferred_element_type=jnp.float32)
    p = jnp.dot(wh, g_ref[...], preferred_element_type=jnp.float32)
    rows = pl.program_id(0) * blk + lax.broadcasted_iota(jnp.int32, (blk, 1), 0)
    par = jnp.bitwise_and(lax.shift_right_logical(rows * 5243, 19), 1)
    p = p + par.astype(jnp.float32) * d_ref[...]
    lo = lax.bitcast_convert_type(p[:, :half].astype(jnp.bfloat16), jnp.uint16)
    hi = lax.bitcast_convert_type(p[:, half:].astype(jnp.bfloat16), jnp.uint16)
    packed = jnp.bitwise_or(lax.shift_left(hi.astype(jnp.int32), 16),
                            lo.astype(jnp.int32))
    out_ref[...] = packed.reshape(out_ref.shape)


def _fin_body(den_ref, num_ref, s_ref, out_ref):
    den = den_ref[0] + den_ref[1]
    num = num_ref[0] + num_ref[1]
    r = num / (den + 1e-16)
    dots = jnp.dot(r, s_ref[...], preferred_element_type=jnp.float32)
    m = jnp.max(dots, axis=1, keepdims=True)
    ex = jnp.exp(dots - m)
    lse = jnp.log(jnp.sum(ex, axis=1, keepdims=True))
    out_ref[...] = dots - m - lse


def _edge_body(src_hbm, dst_hbm, t_hbm, acc_out,
               src_v, dst_v, dst2, i1_2, i2_2, qas, adb, eout,
               acc, sem0, sem1, sem2):
    sems = (sem0, sem1, sem2)
    c = lax.axis_index("c")
    s = lax.axis_index("s")
    zrows = N // NS

    def z_it(j, _):
        eout[j, :] = jnp.zeros((16,), jnp.float32)
        return 0

    lax.fori_loop(0, zrows, z_it, 0)
    pltpu.sync_copy(eout.at[pl.ds(0, zrows)], acc.at[pl.ds(s * zrows, zrows)])
    plsc.subcore_barrier()

    wid = c * NS + s
    lane = lax.iota(jnp.int32, 16)
    lo_half = lane < 8

    def chunk(k, _):
        base = wid * EPW + k * CH
        pltpu.sync_copy(src_hbm.at[pl.ds(base, CH)], src_v)
        pltpu.sync_copy(dst_hbm.at[pl.ds(base, CH)], dst_v)

        def idx_row(j, _):
            def idx_t(t, _):
                fl = pl.ds(j * RB + t * 16, 16)
                sl = pl.ds(t * 16, 16)
                sv = src_v[fl]
                dv = dst_v[fl]
                b2 = lax.shift_right_logical(dv * 5243, 20)
                dst2[j, sl] = dv
                i1_2[j, sl] = sv * ROWS + b2
                i2_2[j, sl] = dv * ROWS + 50
                return 0
            return lax.fori_loop(0, RB // 16, idx_t, 0)

        lax.fori_loop(0, RJ, idx_row, 0)

        def fire(j):
            o = j * RB
            sm = sems[j % 3]
            return (pltpu.async_copy(t_hbm.at[i1_2.at[j]], qas.at[pl.ds(o, RB)], sm),
                    pltpu.async_copy(t_hbm.at[i2_2.at[j]], adb.at[pl.ds(o, RB)], sm))

        def unpk(v):
            lo = lax.bitcast_convert_type(lax.shift_left(v, 16), jnp.float32)
            hi = lax.bitcast_convert_type(jnp.bitwise_and(v, jnp.int32(-65536)),
                                          jnp.float32)
            return lo, hi

        def ee_one(j):
            a16, b16 = unpk(qas[j, :])
            c16, d16 = unpk(adb[j, :])
            sel = a16 + d16 * (b16 - a16)
            e = sel + c16
            e = jnp.maximum(e, e * 0.2)
            ee = jnp.exp(e)
            eer = lax.rev(ee, (0,))
            eout[j, :] = jnp.where(lo_half, eer * sel, ee)

        def ee_it(i, _):
            ee_one(2 * i)
            ee_one(2 * i + 1)
            return 0

        pend = {0: fire(0), 1: fire(1)}
        for j in range(RJ):
            if j + 2 < RJ:
                pend[j + 2] = fire(j + 2)
            for dsc in pend.pop(j):
                dsc.wait()
            lax.fori_loop(j * RB // 2, (j + 1) * RB // 2, ee_it, 0)
            o = j * RB
            pltpu.sync_copy(eout.at[pl.ds(o, RB)], acc.at[dst2.at[j]], add=True)
        return 0

    lax.fori_loop(0, NCHUNK, chunk, 0)

    plsc.subcore_barrier()

    @pl.when(s == 0)
    def _():
        pltpu.sync_copy(acc, acc_out.at[c])


_edge_kernel = functools.partial(
    pl.kernel,
    out_type=jax.ShapeDtypeStruct((NC, N, 16), jnp.float32),
    mesh=plsc.VectorSubcoreMesh(core_axis_name="c", subcore_axis_name="s"),
    compiler_params=pltpu.CompilerParams(use_tc_tiling_on_sc=False),
    scratch_types=[
        pltpu.VMEM((CH,), jnp.int32),
        pltpu.VMEM((CH,), jnp.int32),
        pltpu.VMEM((RJ, RB), jnp.int32),
        pltpu.VMEM((RJ, RB), jnp.int32),
        pltpu.VMEM((RJ, RB), jnp.int32),
        pltpu.VMEM((CH, 16), jnp.int32),
        pltpu.VMEM((CH, 16), jnp.int32),
        pltpu.VMEM((CH, 16), jnp.float32),
        pltpu.VMEM_SHARED((N, 16), jnp.float32),
        pltpu.SemaphoreType.DMA,
        pltpu.SemaphoreType.DMA,
        pltpu.SemaphoreType.DMA,
    ],
)(_edge_body)


def kernel(message, _input, x, edge_index, num_graphs, W, a_src, a_dst, W_fc, b_fc):
    f32 = jnp.float32

    me = pl.pallas_call(
        _me_body,
        out_shape=jax.ShapeDtypeStruct((NG, 128), f32),
    )(message, W_fc, b_fc.reshape(1, 128))

    eyeH = jnp.eye(HEADS, dtype=f32)
    eyeR = eyeH[:, ::-1]
    me3 = me.reshape(NG, HEADS, HDIM)
    Q4 = jnp.einsum('bjd,hj->hdbj', me3, eyeH)
    S3 = jnp.einsum('jd,hj->hdj', a_src[::-1], eyeR)
    D3 = jnp.einsum('jd,hj->hdj', a_dst[::-1], eyeR)
    S4 = jnp.broadcast_to(S3[:, :, None, :], (HEADS, HDIM, NG, HEADS))
    gm = jnp.concatenate([Q4, S4], axis=3).reshape(128, NG, 16)
    A = gm[:, 0::2, :]
    B = gm[:, 1::2, :]
    inter = jnp.stack([A, B], axis=3).reshape(128, 50 * 32)
    c16 = jnp.concatenate([jnp.zeros((HEADS, HDIM, 8), f32), D3],
                          axis=2).reshape(128, 16)
    t2 = jnp.stack([c16, jnp.zeros((128, 16), f32)], axis=2).reshape(128, 32)
    G = jnp.concatenate([inter, t2, jnp.zeros((128, 32), f32)],
                        axis=1)
    G_il = G
    G_even = G_il[:, 0::2]
    G_odd = G_il[:, 1::2]
    zpad = jnp.zeros((128, ROWS * 16 - 832), f32)
    G2 = jnp.concatenate([G_even, zpad, G_odd, zpad], axis=1)
    gcols = jnp.arange(ROWS * 32)
    dsel = ((gcols >= ROWS * 16 + 800) & (gcols < ROWS * 16 + 816)).astype(f32)
    dsel = dsel.reshape(1, ROWS * 32)

    BLK = 400
    T = pl.pallas_call(
        _t_body,
        grid=(N // BLK,),
        in_specs=[
            pl.BlockSpec((BLK, 128), lambda i: (i, 0)),
            pl.BlockSpec((128, 128), lambda i: (0, 0)),
            pl.BlockSpec((128, ROWS * 32), lambda i: (0, 0)),
            pl.BlockSpec((1, ROWS * 32), lambda i: (0, 0)),
        ],
        out_specs=pl.BlockSpec((BLK * ROWS * 16 // 128, 128), lambda i: (i, 0)),
        out_shape=jax.ShapeDtypeStruct((N * ROWS * 16 // 128, 128), jnp.int32),
    )(x, W, G2, dsel)

    t_rows = T.reshape(N * ROWS, 16)

    acc_out = _edge_kernel(edge_index[0], edge_index[1], t_rows)

    num2 = acc_out[:, :, 0:8].reshape(NC, NG, N // NG * HEADS)
    den2 = acc_out[:, :, 8:16][:, :, ::-1].reshape(NC, NG, N // NG * HEADS)
    S = jnp.repeat(jnp.eye(NG, dtype=f32), HEADS, axis=0)

    out = pl.pallas_call(
        _fin_body,
        out_shape=jax.ShapeDtypeStruct((NG, NG), f32),
    )(den2, num2, S)
    return out
